# Initial kernel scaffold; baseline (speedup 1.0000x reference)
import jax
import jax.numpy as jnp
from jax.experimental import pallas as pl


def kernel(tokens, language_ids, embed_weight, lang_weight):
    raise NotImplementedError("write your pallas kernel here")


# speedup vs baseline: 7.4773x; 7.4773x over previous
import functools

import jax
import jax.numpy as jnp
from jax import lax
from jax.experimental import pallas as pl
from jax.experimental.pallas import tpu as pltpu
from jax.experimental.pallas import tpu_sc as plsc

NC = 2
NS = 16
NW = NC * NS
LANES = 16
DEPTH = 4


@functools.cache
def _build(B, T, D, V, NLANG):
    assert B % NW == 0
    nb_w = B // NW
    assert nb_w % DEPTH == 0
    groups = nb_w // DEPTH
    ch_a = min(128, T)
    ch_b = T - ch_a
    assert 0 < ch_b <= 128 and ch_a % 8 == 0 and T % 8 == 0
    ngroups_row = D // LANES

    mesh = plsc.VectorSubcoreMesh(core_axis_name="c", subcore_axis_name="s")

    def body(tok_hbm, lid_hbm, emb_hbm, lang_hbm, out_hbm,
             tok_v, lid_v, lrows_v, b0, b1, b2, b3,
             gs0, gs1, gs2, gs3, os0, os1, os2, os3, lsem):
        bufs = (b0, b1, b2, b3)
        gsems = (gs0, gs1, gs2, gs3)
        osems = (os0, os1, os2, os3)

        cid = lax.axis_index("c")
        sid = lax.axis_index("s")
        wid = sid * NC + cid
        tok_base = wid * (nb_w * T)
        b_base = wid * nb_w

        pltpu.sync_copy(tok_hbm.at[pl.ds(tok_base, nb_w * T)], tok_v)
        pltpu.sync_copy(lid_hbm.at[pl.ds(b_base, nb_w)], lid_v)
        pltpu.async_copy(lang_hbm.at[lid_v], lrows_v, lsem).wait()

        def issue_gather(lb, k):
            s0 = lb * T
            pltpu.async_copy(
                emb_hbm.at[tok_v.at[pl.ds(s0, ch_a)]],
                bufs[k].at[pl.ds(0, ch_a)], gsems[k])
            pltpu.async_copy(
                emb_hbm.at[tok_v.at[pl.ds(s0 + ch_a, ch_b)]],
                bufs[k].at[pl.ds(ch_a, ch_b)], gsems[k])

        def wait_gather(k):
            pltpu.make_async_copy(
                emb_hbm.at[tok_v.at[pl.ds(0, ch_a)]],
                bufs[k].at[pl.ds(0, ch_a)], gsems[k]).wait()
            pltpu.make_async_copy(
                emb_hbm.at[tok_v.at[pl.ds(0, ch_b)]],
                bufs[k].at[pl.ds(ch_a, ch_b)], gsems[k]).wait()

        def issue_out(lb, k):
            row0 = (b_base + lb) * T
            pltpu.async_copy(bufs[k], out_hbm.at[pl.ds(row0, T)], osems[k])

        def wait_out(k):
            pltpu.make_async_copy(
                bufs[k], out_hbm.at[pl.ds(0, T)], osems[k]).wait()

        def add_lang(lb, k):
            buf = bufs[k]
            lvs = [lrows_v[lb, pl.ds(LANES * j, LANES)]
                   for j in range(ngroups_row)]

            def row(r, carry):
                for j in range(ngroups_row):
                    sl = pl.ds(LANES * j, LANES)
                    buf[r, sl] = buf[r, sl] + lvs[j]
                return carry

            lax.fori_loop(0, T, row, 0)

        for k in range(DEPTH):
            issue_gather(k, k)

        def group(g, carry):
            for k in range(DEPTH):
                lb = g * DEPTH + k
                wait_gather(k)
                add_lang(lb, k)
                issue_out(lb, k)

                @pl.when(g < groups - 1)
                def _prefetch(lb=lb, k=k):
                    wait_out(k)
                    issue_gather(lb + DEPTH, k)
            return carry

        lax.fori_loop(0, groups, group, 0)

        for k in range(DEPTH):
            wait_out(k)

    return pl.kernel(
        body,
        out_type=jax.ShapeDtypeStruct((B * T, D), jnp.float32),
        mesh=mesh,
        scratch_types=[
            pltpu.VMEM((nb_w * T,), jnp.int32),
            pltpu.VMEM((nb_w,), jnp.int32),
            pltpu.VMEM((nb_w, D), jnp.float32),
            pltpu.VMEM((T, D), jnp.float32),
            pltpu.VMEM((T, D), jnp.float32),
            pltpu.VMEM((T, D), jnp.float32),
            pltpu.VMEM((T, D), jnp.float32),
        ] + [pltpu.SemaphoreType.DMA] * 9,
    )


def kernel(tokens, language_ids, embed_weight, lang_weight):
    B, T = tokens.shape
    V, D = embed_weight.shape
    tok_flat = tokens.reshape(-1).astype(jnp.int32)
    lid = language_ids.astype(jnp.int32)
    fn = _build(B, T, D, V, lang_weight.shape[0])
    out = fn(tok_flat, lid, embed_weight, lang_weight)
    return out.reshape(B, T, D)
